# baseline (speedup 1.0000x reference)
import jax
import jax.numpy as jnp
from jax import lax
from jax.experimental import pallas as pl
from jax.experimental.pallas import tpu as pltpu
from jax.experimental.pallas import tpu_sc as plsc

NUM_PRIM = 8192
DIM = 64
BATCH = 64
HW = 1024
N = BATCH * HW
NC, NS = 2, 16
NW = NC * NS
CHUNK = 128
HALF_B = BATCH // 2
N_H = N // 2
PER_W = N_H // NW
NCH = PER_W // CHUNK
NBUF = NCH
CSLICE = NUM_PRIM // NS
LANES = 16
HROWS_H = N_H * DIM // 128


def _sc_body(idx_g, table, emb_out, hist_out,
             idx_v, rows_v, ones_v, zeros_v, hist_sh, gsem, ssem, hsem):
    c = lax.axis_index("c")
    s = lax.axis_index("s")
    wid = s * NC + c

    pltpu.sync_copy(idx_g.at[wid], idx_v)

    one = jnp.ones((LANES,), jnp.int32)
    zero = jnp.zeros((LANES,), jnp.int32)
    for i in range(CHUNK // LANES):
        ones_v[pl.ds(i * LANES, LANES)] = one
    for i in range(CSLICE // LANES):
        zeros_v[pl.ds(i * LANES, LANES)] = zero
    pltpu.sync_copy(zeros_v, hist_sh.at[pl.ds(s * CSLICE, CSLICE)])

    plsc.subcore_barrier()

    def hstep(j, carry):
        pltpu.async_copy(ones_v, hist_sh.at[idx_v.at[j]], hsem, add=True)
        return carry
    lax.fori_loop(0, NCH, hstep, 0)

    def prime(k, carry):
        pltpu.async_copy(table.at[idx_v.at[k]], rows_v.at[k], gsem)
        return carry
    lax.fori_loop(0, NBUF, prime, 0)

    def gstep(k, carry):
        pltpu.make_async_copy(table.at[pl.ds(0, CHUNK)],
                              rows_v.at[k], gsem).wait()
        pltpu.async_copy(rows_v.at[k], emb_out.at[wid * NCH + k], ssem)
        return carry
    lax.fori_loop(0, NCH, gstep, 0)

    def sdrain(k, carry):
        pltpu.make_async_copy(table.at[pl.ds(0, CHUNK)],
                              rows_v.at[0], ssem).wait()
        return carry
    lax.fori_loop(0, NCH, sdrain, 0)

    pltpu.make_async_copy(idx_g.at[0], idx_v, hsem).wait()

    plsc.subcore_barrier()

    sl = pl.ds(s * CSLICE, CSLICE)
    pltpu.sync_copy(hist_sh.at[sl], hist_out.at[c].at[sl])


_sc_kernel = pl.kernel(
    _sc_body,
    out_type=(
        jax.ShapeDtypeStruct((NW * NCH, CHUNK, DIM), jnp.float32),
        jax.ShapeDtypeStruct((NC, NUM_PRIM), jnp.int32),
    ),
    mesh=plsc.VectorSubcoreMesh(
        core_axis_name="c", subcore_axis_name="s",
        num_cores=NC, num_subcores=NS,
    ),
    compiler_params=pltpu.CompilerParams(use_tc_tiling_on_sc=False),
    scratch_types=[
        pltpu.VMEM((NCH, CHUNK), jnp.int32),
        pltpu.VMEM((NBUF, CHUNK, DIM), jnp.float32),
        pltpu.VMEM((CHUNK,), jnp.int32),
        pltpu.VMEM((CSLICE,), jnp.int32),
        pltpu.VMEM_SHARED((NUM_PRIM,), jnp.int32),
        pltpu.SemaphoreType.DMA,
        pltpu.SemaphoreType.DMA,
        pltpu.SemaphoreType.DMA,
    ],
)

B_BLK = 16
TC_GRID = HALF_B // B_BLK
ROWS_PER_B = HW * DIM // 128


def _transpose_block(emb_ref, out_ref):
    for bb in range(B_BLK):
        x = emb_ref[pl.ds(bb * ROWS_PER_B, ROWS_PER_B), :]
        out_ref[bb] = jnp.concatenate(
            [x[:, 0:DIM].T, x[:, DIM:2 * DIM].T], axis=1)


def _tc_body_a(emb_ref, out_ref):
    _transpose_block(emb_ref, out_ref)


def _tc_body_b(emb_ref, prev_ref, hist_a_ref, hist_b_ref, act_ref,
               out_ref, cnt_ref):
    del prev_ref
    _transpose_block(emb_ref, out_ref)

    @pl.when(pl.program_id(0) == 0)
    def _():
        cnt_ref[...] = (act_ref[...]
                        + hist_a_ref[0] + hist_a_ref[1]
                        + hist_b_ref[0] + hist_b_ref[1])


_tc_kernel_a = pl.pallas_call(
    _tc_body_a,
    grid=(TC_GRID,),
    in_specs=[pl.BlockSpec((B_BLK * ROWS_PER_B, 128), lambda i: (i, 0))],
    out_specs=pl.BlockSpec((B_BLK, DIM, HW), lambda i: (i, 0, 0)),
    out_shape=jax.ShapeDtypeStruct((BATCH, DIM, HW), jnp.float32),
)

_tc_kernel_b = pl.pallas_call(
    _tc_body_b,
    grid=(TC_GRID,),
    in_specs=[
        pl.BlockSpec((B_BLK * ROWS_PER_B, 128), lambda i: (i, 0)),
        pl.BlockSpec((1, DIM, 128), lambda i: (0, 0, 0)),
        pl.BlockSpec((NC, 64, 128), lambda i: (0, 0, 0)),
        pl.BlockSpec((NC, 64, 128), lambda i: (0, 0, 0)),
        pl.BlockSpec((64, 128), lambda i: (0, 0)),
    ],
    out_specs=[
        pl.BlockSpec((B_BLK, DIM, HW), lambda i: (i + TC_GRID, 0, 0)),
        pl.BlockSpec((64, 128), lambda i: (0, 0)),
    ],
    out_shape=(
        jax.ShapeDtypeStruct((BATCH, DIM, HW), jnp.float32),
        jax.ShapeDtypeStruct((64, 128), jnp.int32),
    ),
    input_output_aliases={1: 0},
)


@jax.jit
def kernel(indices, primitives, activation_count):
    hh = jnp.arange(HW, dtype=jnp.int32)
    perm = (hh % 2) * (HW // 2) + hh // 2
    idx_perm = jnp.take(indices, perm, axis=1)
    idx_a = idx_perm[:HALF_B].reshape(NW, NCH, CHUNK)
    idx_b = idx_perm[HALF_B:].reshape(NW, NCH, CHUNK)

    emb_a, hist_a = _sc_kernel(idx_a, primitives)
    emb_b, hist_b = _sc_kernel(idx_b, primitives)

    part = _tc_kernel_a(emb_a.reshape(HROWS_H, 128))
    out_t, cnt2d = _tc_kernel_b(
        emb_b.reshape(HROWS_H, 128),
        part,
        hist_a.reshape(NC, 64, 128),
        hist_b.reshape(NC, 64, 128),
        activation_count.reshape(64, 128),
    )
    return out_t.transpose(0, 2, 1), cnt2d.reshape(NUM_PRIM)
